# baseline (speedup 1.0000x reference)
import functools

import jax
import jax.numpy as jnp
from jax import lax
from jax.experimental import pallas as pl
from jax.experimental.pallas import tpu as pltpu
from jax.experimental.pallas import tpu_sc as plsc

N = 10000
NE = 320000
D = 128

NC = 2
NS = 16
NW = NC * NS

EPW = NE // NW

B = 128
NB0 = 160
NB1 = 0
CH = 8
TOT_B = NS * (NB0 + NB1)
SLOTS = TOT_B * B
ACC_ROWS = 10240
RPT = ACC_ROWS // NS
TRASH0 = N + 16

_MESH = plsc.VectorSubcoreMesh(core_axis_name="c", subcore_axis_name="s")


@functools.partial(
    pl.kernel,
    out_type=jax.ShapeDtypeStruct((NW, N), jnp.float32),
    mesh=_MESH,
    compiler_params=pltpu.CompilerParams(needs_layout_passes=False),
    scratch_types=[
        pltpu.VMEM((EPW,), jnp.int32),
        pltpu.VMEM((N,), jnp.float32),
    ],
)
def _deg_kernel(dst_hbm, part_hbm, idx_v, hist_v):
    cid = lax.axis_index("c")
    sid = lax.axis_index("s")
    wid = sid * NC + cid

    pltpu.sync_copy(dst_hbm.at[pl.ds(wid * EPW, EPW)], idx_v)

    zeros = jnp.zeros((16,), jnp.float32)

    def _zero(i, carry):
        hist_v[pl.ds(i * 16, 16)] = zeros
        return carry

    lax.fori_loop(0, N // 16, _zero, 0)

    ones = jnp.ones((16,), jnp.float32)

    def _scat(i, carry):
        idx = idx_v[pl.ds(i * 16, 16)]
        plsc.addupdate_scatter(hist_v, [idx], ones)
        return carry

    lax.fori_loop(0, EPW // 16, _scat, 0)

    pltpu.sync_copy(hist_v, part_hbm.at[wid])


def _dense_body(x_ref, ws_ref, wc_ref, bs_ref, bc_ref, degp_ref, hp_ref, base_ref):
    xb = x_ref[...]
    deg = jnp.sum(degp_ref[0], axis=0) + 1.0
    dinv = lax.rsqrt(deg)
    h = jnp.dot(xb, wc_ref[...], preferred_element_type=jnp.float32)
    hp = h * dinv[:, None]
    base = (
        jnp.dot(xb, ws_ref[...], preferred_element_type=jnp.float32)
        + bs_ref[...]
        + bc_ref[...]
        + hp * dinv[:, None]
    )
    hp_ref[...] = hp
    base_ref[...] = base


_R = 1000


def _dense(x, W_self, W_conv, b_self, b_conv, degp):
    return pl.pallas_call(
        _dense_body,
        grid=(N // _R,),
        in_specs=[
            pl.BlockSpec((_R, D), lambda i: (i, 0)),
            pl.BlockSpec((D, D), lambda i: (0, 0)),
            pl.BlockSpec((D, D), lambda i: (0, 0)),
            pl.BlockSpec((1, D), lambda i: (0, 0)),
            pl.BlockSpec((1, D), lambda i: (0, 0)),
            pl.BlockSpec((1, NW, _R), lambda i: (i, 0, 0)),
        ],
        out_specs=[
            pl.BlockSpec((_R, D), lambda i: (i, 0)),
            pl.BlockSpec((_R, D), lambda i: (i, 0)),
        ],
        out_shape=[
            jax.ShapeDtypeStruct((N, D), jnp.float32),
            jax.ShapeDtypeStruct((N, D), jnp.float32),
        ],
    )(x, W_self, W_conv, b_self, b_conv, degp)


@functools.partial(
    pl.kernel,
    out_type=jax.ShapeDtypeStruct((NC, ACC_ROWS, D), jnp.float32),
    mesh=_MESH,
    scratch_types=[
        pltpu.VMEM((CH, 2, B), jnp.int32),
        pltpu.VMEM((CH, 2, B), jnp.int32),
        pltpu.VMEM((B, D), jnp.float32),
        pltpu.VMEM((B, D), jnp.float32),
        pltpu.VMEM((16, D), jnp.float32),
        pltpu.VMEM_SHARED((ACC_ROWS, D), jnp.float32),
        pltpu.SemaphoreType.DMA,
        pltpu.SemaphoreType.DMA,
        pltpu.SemaphoreType.DMA,
        pltpu.SemaphoreType.DMA,
    ],
)
def _edge_kernel(hp_hbm, idx_hbm, part_hbm,
                 ich0_v, ich1_v, rows0_v, rows1_v, zst_v, acc_sh,
                 semr0, semr1, semi0, semi1):
    cid = lax.axis_index("c")
    sid = lax.axis_index("s")

    nb = jnp.where(cid == 0, NB0, NB1)
    nch = nb // CH
    base_b = jnp.where(cid == 0, sid * NB0, NS * NB0 + sid * NB1)

    zeros = jnp.zeros((16,), jnp.float32)

    def _zrow(i, carry):
        def _zcol(c, carry2):
            zst_v[i, pl.ds(c * 16, 16)] = zeros
            return carry2

        return lax.fori_loop(0, D // 16, _zcol, carry)

    lax.fori_loop(0, 16, _zrow, 0)

    def _zacc(j, carry):
        pltpu.sync_copy(zst_v, acc_sh.at[pl.ds(sid * RPT + j * 16, 16)])
        return carry

    lax.fori_loop(0, RPT // 16, _zacc, 0)

    plsc.subcore_barrier()

    ich = (ich0_v, ich1_v)
    semi = (semi0, semi1)
    rows = (rows0_v, rows1_v)
    semr = (semr0, semr1)
    my_hp = hp_hbm.at[cid]

    def _load_chunk(c, k):
        pltpu.async_copy(idx_hbm.at[pl.ds((base_b + c * CH) * 1, CH)],
                         ich[k], semi[k])

    def _wait_chunk(c, k):
        pltpu.make_async_copy(idx_hbm.at[pl.ds((base_b + c * CH) * 1, CH)],
                              ich[k], semi[k]).wait()

    @pl.when(nb > 0)
    def _prologue():
        _load_chunk(0, 0)
        _load_chunk(1, 1)
        _wait_chunk(0, 0)
        pltpu.async_copy(my_hp.at[ich0_v.at[0, 0]], rows0_v, semr0)
        pltpu.async_copy(my_hp.at[ich0_v.at[1, 0]], rows1_v, semr1)

    def _pair(p, carry):
        b = 2 * p
        c = lax.div(b, CH)
        ci = lax.rem(c, 2)
        bo = lax.rem(b, CH)
        last_pair = bo == CH - 2

        @pl.when(last_pair & (c + 1 < nch))
        def _wait_next():
            for k in range(2):
                @pl.when(ci == k)
                def _w():
                    _wait_chunk(c + 1, 1 - k)

        for half in range(2):
            bb = b + half

            for kc in range(2):
                @pl.when(ci == kc)
                def _drain():
                    pltpu.make_async_copy(
                        my_hp.at[ich[kc].at[bo + half, 0]],
                        rows[half], semr[half]).wait()
                    pltpu.sync_copy(rows[half],
                                    acc_sh.at[ich[kc].at[bo + half, 1]],
                                    add=True)

                    @pl.when(bb + 2 < nb)
                    def _next_gather():
                        @pl.when(jnp.logical_not(last_pair))
                        def _same():
                            pltpu.async_copy(
                                my_hp.at[ich[kc].at[bo + half + 2, 0]],
                                rows[half], semr[half])

                        @pl.when(last_pair)
                        def _next():
                            pltpu.async_copy(
                                my_hp.at[ich[1 - kc].at[half, 0]],
                                rows[half], semr[half])

        @pl.when(last_pair & (c + 2 < nch))
        def _next_chunk():
            for k in range(2):
                @pl.when(ci == k)
                def _pf():
                    _load_chunk(c + 2, k)

        return carry

    lax.fori_loop(0, nb // 2, _pair, 0)

    plsc.subcore_barrier()

    pltpu.sync_copy(acc_sh.at[pl.ds(sid * RPT, RPT)],
                    part_hbm.at[cid].at[pl.ds(sid * RPT, RPT)])


def _combine_body(base_ref, parts_ref, degp_ref, out_ref):
    deg = jnp.sum(degp_ref[0], axis=0) + 1.0
    dinv = lax.rsqrt(deg)
    s = parts_ref[0] + parts_ref[1]
    out_ref[...] = base_ref[...] + dinv[:, None] * s


def _combine(base, parts, degp):
    return pl.pallas_call(
        _combine_body,
        grid=(N // _R,),
        in_specs=[
            pl.BlockSpec((_R, D), lambda i: (i, 0)),
            pl.BlockSpec((NC, _R, D), lambda i: (0, i, 0)),
            pl.BlockSpec((1, NW, _R), lambda i: (i, 0, 0)),
        ],
        out_specs=pl.BlockSpec((_R, D), lambda i: (i, 0)),
        out_shape=jax.ShapeDtypeStruct((N, D), jnp.float32),
    )(base, parts, degp)


def kernel(x, edge_index, n1, n2, W_self, b_self, W_conv, b_conv):
    ei = edge_index.astype(jnp.int32)
    src = ei[0]
    dst = ei[1]

    degp = _deg_kernel(dst)
    degp_b = degp.reshape(NW, N // _R, _R).swapaxes(0, 1)

    hp, base = _dense(x, W_self, W_conv,
                      b_self.reshape(1, D), b_conv.reshape(1, D), degp_b)

    pad = SLOTS - NE
    src_p = jnp.concatenate([src, jnp.zeros((pad,), jnp.int32)]).reshape(TOT_B, B)
    trash = TRASH0 + (jnp.arange(pad, dtype=jnp.int32) % (ACC_ROWS - TRASH0))
    dst_p = jnp.concatenate([dst, trash]).reshape(TOT_B, B)
    idx = jnp.stack([src_p, dst_p], axis=1)

    hp2 = jnp.stack([hp, hp])
    parts = _edge_kernel(hp2, idx)

    return _combine(base, parts, degp_b)


# speedup vs baseline: 3.6165x; 3.6165x over previous
import functools

import jax
import jax.numpy as jnp
from jax import lax
from jax.experimental import pallas as pl
from jax.experimental.pallas import tpu as pltpu
from jax.experimental.pallas import tpu_sc as plsc

N = 10000
NE = 320000
D = 128

NC = 2
NS = 16
NW = NC * NS

EPW = NE // NW

B = 128
NB0 = 80
NB1 = 80
CH = 8
TOT_B = NS * (NB0 + NB1)
SLOTS = TOT_B * B
ACC_ROWS = 10240
RPT = ACC_ROWS // NS
TRASH0 = N + 16

_MESH = plsc.VectorSubcoreMesh(core_axis_name="c", subcore_axis_name="s")


@functools.partial(
    pl.kernel,
    out_type=jax.ShapeDtypeStruct((NW, N), jnp.float32),
    mesh=_MESH,
    compiler_params=pltpu.CompilerParams(needs_layout_passes=False),
    scratch_types=[
        pltpu.VMEM((EPW,), jnp.int32),
        pltpu.VMEM((N,), jnp.float32),
    ],
)
def _deg_kernel(dst_hbm, part_hbm, idx_v, hist_v):
    cid = lax.axis_index("c")
    sid = lax.axis_index("s")
    wid = sid * NC + cid

    pltpu.sync_copy(dst_hbm.at[pl.ds(wid * EPW, EPW)], idx_v)

    zeros = jnp.zeros((16,), jnp.float32)

    def _zero(i, carry):
        hist_v[pl.ds(i * 16, 16)] = zeros
        return carry

    lax.fori_loop(0, N // 16, _zero, 0)

    ones = jnp.ones((16,), jnp.float32)

    def _scat(i, carry):
        idx = idx_v[pl.ds(i * 16, 16)]
        plsc.addupdate_scatter(hist_v, [idx], ones)
        return carry

    lax.fori_loop(0, EPW // 16, _scat, 0)

    pltpu.sync_copy(hist_v, part_hbm.at[wid])


def _dense_body(x_ref, ws_ref, wc_ref, bs_ref, bc_ref, degp_ref, hp_ref, base_ref):
    xb = x_ref[...]
    deg = jnp.sum(degp_ref[0], axis=0) + 1.0
    dinv = lax.rsqrt(deg)
    h = jnp.dot(xb, wc_ref[...], preferred_element_type=jnp.float32)
    hp = h * dinv[:, None]
    base = (
        jnp.dot(xb, ws_ref[...], preferred_element_type=jnp.float32)
        + bs_ref[...]
        + bc_ref[...]
        + hp * dinv[:, None]
    )
    hp_ref[...] = hp
    base_ref[...] = base


_R = 1000


def _dense(x, W_self, W_conv, b_self, b_conv, degp):
    return pl.pallas_call(
        _dense_body,
        grid=(N // _R,),
        in_specs=[
            pl.BlockSpec((_R, D), lambda i: (i, 0)),
            pl.BlockSpec((D, D), lambda i: (0, 0)),
            pl.BlockSpec((D, D), lambda i: (0, 0)),
            pl.BlockSpec((1, D), lambda i: (0, 0)),
            pl.BlockSpec((1, D), lambda i: (0, 0)),
            pl.BlockSpec((1, NW, _R), lambda i: (i, 0, 0)),
        ],
        out_specs=[
            pl.BlockSpec((_R, D), lambda i: (i, 0)),
            pl.BlockSpec((_R, D), lambda i: (i, 0)),
        ],
        out_shape=[
            jax.ShapeDtypeStruct((N, D), jnp.float32),
            jax.ShapeDtypeStruct((N, D), jnp.float32),
        ],
    )(x, W_self, W_conv, b_self, b_conv, degp)


@functools.partial(
    pl.kernel,
    out_type=jax.ShapeDtypeStruct((NC, ACC_ROWS, D), jnp.float32),
    mesh=_MESH,
    scratch_types=[
        pltpu.VMEM((CH, 2, B), jnp.int32),
        pltpu.VMEM((CH, 2, B), jnp.int32),
        pltpu.VMEM((B, D), jnp.float32),
        pltpu.VMEM((B, D), jnp.float32),
        pltpu.VMEM((16, D), jnp.float32),
        pltpu.VMEM_SHARED((ACC_ROWS, D), jnp.float32),
        pltpu.SemaphoreType.DMA,
        pltpu.SemaphoreType.DMA,
        pltpu.SemaphoreType.DMA,
        pltpu.SemaphoreType.DMA,
    ],
)
def _edge_kernel(hp_hbm, idx_hbm, part_hbm,
                 ich0_v, ich1_v, rows0_v, rows1_v, zst_v, acc_sh,
                 semr0, semr1, semi0, semi1):
    cid = lax.axis_index("c")
    sid = lax.axis_index("s")

    nb = jnp.where(cid == 0, NB0, NB1)
    nch = nb // CH
    base_b = jnp.where(cid == 0, sid * NB0, NS * NB0 + sid * NB1)

    zeros = jnp.zeros((16,), jnp.float32)

    def _zrow(i, carry):
        def _zcol(c, carry2):
            zst_v[i, pl.ds(c * 16, 16)] = zeros
            return carry2

        return lax.fori_loop(0, D // 16, _zcol, carry)

    lax.fori_loop(0, 16, _zrow, 0)

    def _zacc(j, carry):
        pltpu.sync_copy(zst_v, acc_sh.at[pl.ds(sid * RPT + j * 16, 16)])
        return carry

    lax.fori_loop(0, RPT // 16, _zacc, 0)

    plsc.subcore_barrier()

    ich = (ich0_v, ich1_v)
    semi = (semi0, semi1)
    rows = (rows0_v, rows1_v)
    semr = (semr0, semr1)
    my_hp = hp_hbm.at[cid]

    def _load_chunk(c, k):
        pltpu.async_copy(idx_hbm.at[pl.ds((base_b + c * CH) * 1, CH)],
                         ich[k], semi[k])

    def _wait_chunk(c, k):
        pltpu.make_async_copy(idx_hbm.at[pl.ds((base_b + c * CH) * 1, CH)],
                              ich[k], semi[k]).wait()

    @pl.when(nb > 0)
    def _prologue():
        _load_chunk(0, 0)
        _load_chunk(1, 1)
        _wait_chunk(0, 0)
        pltpu.async_copy(my_hp.at[ich0_v.at[0, 0]], rows0_v, semr0)
        pltpu.async_copy(my_hp.at[ich0_v.at[1, 0]], rows1_v, semr1)

    def _pair(p, carry):
        b = 2 * p
        c = lax.div(b, CH)
        ci = lax.rem(c, 2)
        bo = lax.rem(b, CH)
        last_pair = bo == CH - 2

        @pl.when(last_pair & (c + 1 < nch))
        def _wait_next():
            for k in range(2):
                @pl.when(ci == k)
                def _w():
                    _wait_chunk(c + 1, 1 - k)

        for half in range(2):
            bb = b + half

            for kc in range(2):
                @pl.when(ci == kc)
                def _drain():
                    pltpu.make_async_copy(
                        my_hp.at[ich[kc].at[bo + half, 0]],
                        rows[half], semr[half]).wait()
                    pltpu.sync_copy(rows[half],
                                    acc_sh.at[ich[kc].at[bo + half, 1]],
                                    add=True)

                    @pl.when(bb + 2 < nb)
                    def _next_gather():
                        @pl.when(jnp.logical_not(last_pair))
                        def _same():
                            pltpu.async_copy(
                                my_hp.at[ich[kc].at[bo + half + 2, 0]],
                                rows[half], semr[half])

                        @pl.when(last_pair)
                        def _next():
                            pltpu.async_copy(
                                my_hp.at[ich[1 - kc].at[half, 0]],
                                rows[half], semr[half])

        @pl.when(last_pair & (c + 2 < nch))
        def _next_chunk():
            for k in range(2):
                @pl.when(ci == k)
                def _pf():
                    _load_chunk(c + 2, k)

        return carry

    lax.fori_loop(0, nb // 2, _pair, 0)

    plsc.subcore_barrier()

    pltpu.sync_copy(acc_sh.at[pl.ds(sid * RPT, RPT)],
                    part_hbm.at[cid].at[pl.ds(sid * RPT, RPT)])


def _combine_body(base_ref, parts_ref, degp_ref, out_ref):
    deg = jnp.sum(degp_ref[0], axis=0) + 1.0
    dinv = lax.rsqrt(deg)
    s = parts_ref[0] + parts_ref[1]
    out_ref[...] = base_ref[...] + dinv[:, None] * s


def _combine(base, parts, degp):
    return pl.pallas_call(
        _combine_body,
        grid=(N // _R,),
        in_specs=[
            pl.BlockSpec((_R, D), lambda i: (i, 0)),
            pl.BlockSpec((NC, _R, D), lambda i: (0, i, 0)),
            pl.BlockSpec((1, NW, _R), lambda i: (i, 0, 0)),
        ],
        out_specs=pl.BlockSpec((_R, D), lambda i: (i, 0)),
        out_shape=jax.ShapeDtypeStruct((N, D), jnp.float32),
    )(base, parts, degp)


def kernel(x, edge_index, n1, n2, W_self, b_self, W_conv, b_conv):
    ei = edge_index.astype(jnp.int32)
    src = ei[0]
    dst = ei[1]

    degp = _deg_kernel(dst)
    degp_b = degp.reshape(NW, N // _R, _R).swapaxes(0, 1)

    hp, base = _dense(x, W_self, W_conv,
                      b_self.reshape(1, D), b_conv.reshape(1, D), degp_b)

    pad = SLOTS - NE
    pad_src = jnp.arange(pad, dtype=jnp.int32) % N
    src_p = jnp.concatenate([src, pad_src]).reshape(TOT_B, B)
    trash = TRASH0 + (jnp.arange(pad, dtype=jnp.int32) % (ACC_ROWS - TRASH0))
    dst_p = jnp.concatenate([dst, trash]).reshape(TOT_B, B)
    idx = jnp.stack([src_p, dst_p], axis=1)

    hp2 = jnp.stack([hp, hp])
    parts = _edge_kernel(hp2, idx)

    return _combine(base, parts, degp_b)


# speedup vs baseline: 3.7938x; 1.0490x over previous
import functools

import jax
import jax.numpy as jnp
from jax import lax
from jax.experimental import pallas as pl
from jax.experimental.pallas import tpu as pltpu
from jax.experimental.pallas import tpu_sc as plsc

N = 10000
NE = 320000
D = 128

NC = 2
NS = 16
NW = NC * NS

EPW = NE // NW

B = 128
NB0 = 80
NB1 = 80
CH = 8
TOT_B = NS * (NB0 + NB1)
SLOTS = TOT_B * B
ACC_ROWS = 10240
RPT = ACC_ROWS // NS
TRASH0 = N + 16

_MESH = plsc.VectorSubcoreMesh(core_axis_name="c", subcore_axis_name="s")


@functools.partial(
    pl.kernel,
    out_type=jax.ShapeDtypeStruct((NW, N), jnp.float32),
    mesh=_MESH,
    compiler_params=pltpu.CompilerParams(needs_layout_passes=False),
    scratch_types=[
        pltpu.VMEM((EPW,), jnp.int32),
        pltpu.VMEM((N,), jnp.float32),
    ],
)
def _deg_kernel(dst_hbm, part_hbm, idx_v, hist_v):
    cid = lax.axis_index("c")
    sid = lax.axis_index("s")
    wid = sid * NC + cid

    pltpu.sync_copy(dst_hbm.at[pl.ds(wid * EPW, EPW)], idx_v)

    zeros = jnp.zeros((16,), jnp.float32)

    def _zero(i, carry):
        hist_v[pl.ds(i * 16, 16)] = zeros
        return carry

    lax.fori_loop(0, N // 16, _zero, 0)

    ones = jnp.ones((16,), jnp.float32)

    def _scat(i, carry):
        idx = idx_v[pl.ds(i * 16, 16)]
        plsc.addupdate_scatter(hist_v, [idx], ones)
        return carry

    lax.fori_loop(0, EPW // 16, _scat, 0)

    pltpu.sync_copy(hist_v, part_hbm.at[wid])


def _dense_body(x_ref, ws_ref, wc_ref, bs_ref, bc_ref, degp_ref, hp_ref, base_ref):
    xb = x_ref[...]
    deg = jnp.sum(degp_ref[0], axis=0) + 1.0
    dinv = lax.rsqrt(deg)
    h = jnp.dot(xb, wc_ref[...], preferred_element_type=jnp.float32)
    hp = h * dinv[:, None]
    base = (
        jnp.dot(xb, ws_ref[...], preferred_element_type=jnp.float32)
        + bs_ref[...]
        + bc_ref[...]
        + hp * dinv[:, None]
    )
    hp_ref[...] = hp
    base_ref[...] = base


_R = 1000


def _dense(x, W_self, W_conv, b_self, b_conv, degp):
    return pl.pallas_call(
        _dense_body,
        grid=(N // _R,),
        in_specs=[
            pl.BlockSpec((_R, D), lambda i: (i, 0)),
            pl.BlockSpec((D, D), lambda i: (0, 0)),
            pl.BlockSpec((D, D), lambda i: (0, 0)),
            pl.BlockSpec((1, D), lambda i: (0, 0)),
            pl.BlockSpec((1, D), lambda i: (0, 0)),
            pl.BlockSpec((1, NW, _R), lambda i: (i, 0, 0)),
        ],
        out_specs=[
            pl.BlockSpec((_R, D), lambda i: (i, 0)),
            pl.BlockSpec((_R, D), lambda i: (i, 0)),
        ],
        out_shape=[
            jax.ShapeDtypeStruct((N, D), jnp.float32),
            jax.ShapeDtypeStruct((N, D), jnp.float32),
        ],
    )(x, W_self, W_conv, b_self, b_conv, degp)


@functools.partial(
    pl.kernel,
    out_type=jax.ShapeDtypeStruct((NC, ACC_ROWS, D), jnp.float32),
    mesh=_MESH,
    scratch_types=[
        pltpu.VMEM((CH, B), jnp.int32),
        pltpu.VMEM((CH, B), jnp.int32),
        pltpu.VMEM((CH, B), jnp.int32),
        pltpu.VMEM((CH, B), jnp.int32),
        pltpu.VMEM((B, D), jnp.float32),
        pltpu.VMEM((B, D), jnp.float32),
        pltpu.VMEM((16, D), jnp.float32),
        pltpu.VMEM_SHARED((ACC_ROWS, D), jnp.float32),
        pltpu.SemaphoreType.DMA,
        pltpu.SemaphoreType.DMA,
        pltpu.SemaphoreType.DMA,
        pltpu.SemaphoreType.DMA,
    ],
)
def _edge_kernel(hp_hbm, idx_hbm, part_hbm,
                 ichs0_v, ichs1_v, ichd0_v, ichd1_v, rows0_v, rows1_v,
                 zst_v, acc_sh, semr0, semr1, semi0, semi1):
    cid = lax.axis_index("c")
    sid = lax.axis_index("s")

    nb = jnp.where(cid == 0, NB0, NB1)
    nch = nb // CH
    base_b = jnp.where(cid == 0, sid * NB0, NS * NB0 + sid * NB1)

    zeros = jnp.zeros((16,), jnp.float32)

    def _zrow(i, carry):
        def _zcol(c, carry2):
            zst_v[i, pl.ds(c * 16, 16)] = zeros
            return carry2

        return lax.fori_loop(0, D // 16, _zcol, carry)

    lax.fori_loop(0, 16, _zrow, 0)

    def _zacc(j, carry):
        pltpu.sync_copy(zst_v, acc_sh.at[pl.ds(sid * RPT + j * 16, 16)])
        return carry

    lax.fori_loop(0, RPT // 16, _zacc, 0)

    plsc.subcore_barrier()

    ichs = (ichs0_v, ichs1_v)
    ichd = (ichd0_v, ichd1_v)
    semi = (semi0, semi1)
    rows = (rows0_v, rows1_v)
    semr = (semr0, semr1)
    my_hp = hp_hbm

    def _load_chunk(c, k):
        sl = pl.ds(base_b + c * CH, CH)
        pltpu.async_copy(idx_hbm.at[0].at[sl], ichs[k], semi[k])
        pltpu.async_copy(idx_hbm.at[1].at[sl], ichd[k], semi[k])

    def _wait_chunk(c, k):
        sl = pl.ds(base_b + c * CH, CH)
        pltpu.make_async_copy(idx_hbm.at[0].at[sl], ichs[k], semi[k]).wait()
        pltpu.make_async_copy(idx_hbm.at[1].at[sl], ichd[k], semi[k]).wait()

    @pl.when(nb > 0)
    def _prologue():
        _load_chunk(0, 0)
        _load_chunk(1, 1)
        _wait_chunk(0, 0)
        pltpu.async_copy(my_hp.at[ichs0_v.at[0]], rows0_v, semr0)
        pltpu.async_copy(my_hp.at[ichs0_v.at[1]], rows1_v, semr1)

    def _pair(p, carry):
        b = 2 * p
        c = lax.div(b, CH)
        ci = lax.rem(c, 2)
        bo = lax.rem(b, CH)
        last_pair = bo == CH - 2

        @pl.when(last_pair & (c + 1 < nch))
        def _wait_next():
            for k in range(2):
                @pl.when(ci == k)
                def _w():
                    _wait_chunk(c + 1, 1 - k)

        for half in range(2):
            bb = b + half

            for kc in range(2):
                @pl.when(ci == kc)
                def _drain():
                    pltpu.make_async_copy(
                        my_hp.at[ichs[kc].at[bo + half]],
                        rows[half], semr[half]).wait()
                    pltpu.sync_copy(rows[half],
                                    acc_sh.at[ichd[kc].at[bo + half]],
                                    add=True)

                    @pl.when(bb + 2 < nb)
                    def _next_gather():
                        @pl.when(jnp.logical_not(last_pair))
                        def _same():
                            pltpu.async_copy(
                                my_hp.at[ichs[kc].at[bo + half + 2]],
                                rows[half], semr[half])

                        @pl.when(last_pair)
                        def _next():
                            pltpu.async_copy(
                                my_hp.at[ichs[1 - kc].at[half]],
                                rows[half], semr[half])

        @pl.when(last_pair & (c + 2 < nch))
        def _next_chunk():
            for k in range(2):
                @pl.when(ci == k)
                def _pf():
                    _load_chunk(c + 2, k)

        return carry

    lax.fori_loop(0, nb // 2, _pair, 0)

    plsc.subcore_barrier()

    pltpu.sync_copy(acc_sh.at[pl.ds(sid * RPT, RPT)],
                    part_hbm.at[cid].at[pl.ds(sid * RPT, RPT)])


def _combine_body(base_ref, parts_ref, degp_ref, out_ref):
    deg = jnp.sum(degp_ref[0], axis=0) + 1.0
    dinv = lax.rsqrt(deg)
    s = parts_ref[0] + parts_ref[1]
    out_ref[...] = base_ref[...] + dinv[:, None] * s


def _combine(base, parts, degp):
    return pl.pallas_call(
        _combine_body,
        grid=(N // _R,),
        in_specs=[
            pl.BlockSpec((_R, D), lambda i: (i, 0)),
            pl.BlockSpec((NC, _R, D), lambda i: (0, i, 0)),
            pl.BlockSpec((1, NW, _R), lambda i: (i, 0, 0)),
        ],
        out_specs=pl.BlockSpec((_R, D), lambda i: (i, 0)),
        out_shape=jax.ShapeDtypeStruct((N, D), jnp.float32),
    )(base, parts, degp)


def kernel(x, edge_index, n1, n2, W_self, b_self, W_conv, b_conv):
    ei = edge_index.astype(jnp.int32)
    src = ei[0]
    dst = ei[1]

    degp = _deg_kernel(dst)
    degp_b = degp.reshape(NW, N // _R, _R).swapaxes(0, 1)

    hp, base = _dense(x, W_self, W_conv,
                      b_self.reshape(1, D), b_conv.reshape(1, D), degp_b)

    pad = SLOTS - NE
    pad_src = jnp.arange(pad, dtype=jnp.int32) % N
    trash = TRASH0 + (jnp.arange(pad, dtype=jnp.int32) % (ACC_ROWS - TRASH0))
    idx = jnp.concatenate([ei, jnp.stack([pad_src, trash])], axis=1)
    idx = idx.reshape(2, TOT_B, B)

    parts = _edge_kernel(hp, idx)

    return _combine(base, parts, degp_b)
